# baseline (speedup 1.0000x reference)
import functools

import jax
import jax.numpy as jnp
from jax import lax
from jax.experimental import pallas as pl
from jax.experimental.pallas import tpu as pltpu
from jax.experimental.pallas import tpu_sc as plsc

N = 10000
E = 320000
D = 128
G = 64
C = 10

NC = 2
NS = 16
NW = NC * NS
EPW = E // NW
CH = 128
EPWP = 10240
NCHUNK = EPWP // CH

NP = 10240
NPT = NP // NS
BLK = 512
NBLK = NP // BLK

_f32 = jnp.float32
_i32 = jnp.int32

_MESH = plsc.VectorSubcoreMesh(core_axis_name="c", subcore_axis_name="s")

_SC_PARAMS = pltpu.CompilerParams()
if "needs_layout_passes" in pltpu.CompilerParams.__dataclass_fields__:
  import dataclasses as _dataclasses
  _SC_PARAMS = _dataclasses.replace(_SC_PARAMS, needs_layout_passes=False)


H8 = NP * 8


@functools.partial(
    pl.kernel,
    out_type=jax.ShapeDtypeStruct((NW, H8), _f32),
    mesh=_MESH,
    scratch_types=[
        pltpu.VMEM((NCHUNK, CH), _i32),
        pltpu.VMEM((2, 16), _i32),
        pltpu.VMEM((H8,), _f32),
    ],
    compiler_params=_SC_PARAMS,
)
def _sc_degree(aux_hbm, dstr_hbm, out_hbm, dst_v, aux_v, hist_v):
  c = lax.axis_index("c")
  s = lax.axis_index("s")
  w = c * NS + s
  pltpu.sync_copy(dstr_hbm.at[w], dst_v)
  pltpu.sync_copy(aux_hbm, aux_v)

  zeros16 = jnp.zeros((16,), _f32)
  ones16 = jnp.full((16,), 1.0, _f32)

  @pl.loop(0, H8 // 16)
  def _(r):
    hist_v[pl.ds(r * 16, 16)] = zeros16

  lane = aux_v[0]
  offs = aux_v[1]
  mask_lo = lane < 8
  mask_hi = lane >= 8

  @pl.loop(0, NCHUNK)
  def _(j):
    @pl.loop(0, CH // 16)
    def _(l):
      d = dst_v[j, pl.ds(l * 16, 16)]
      addr = d * 8 + offs
      plsc.addupdate_scatter(hist_v, [addr], ones16, mask=mask_lo)
      plsc.addupdate_scatter(hist_v, [addr], ones16, mask=mask_hi)

  pltpu.sync_copy(hist_v, out_hbm.at[w])


@functools.partial(
    pl.kernel,
    out_type=jax.ShapeDtypeStruct((NW, NPT, D), _f32),
    mesh=_MESH,
    scratch_types=[
        pltpu.VMEM((NCHUNK, CH), _i32),
        pltpu.VMEM((NCHUNK, CH), _i32),
        pltpu.VMEM((CH, D), _f32),
        pltpu.VMEM_SHARED((NP, D), _f32),
    ],
)
def _sc_aggregate(g_hbm, z_hbm, srcr_hbm, dstr_hbm, out_hbm, src_v, dst_v,
                  rows_v, acc):
  c = lax.axis_index("c")
  s = lax.axis_index("s")
  w = c * NS + s
  pltpu.sync_copy(srcr_hbm.at[w], src_v)
  pltpu.sync_copy(dstr_hbm.at[w], dst_v)

  pltpu.sync_copy(z_hbm, acc.at[pl.ds(s * NPT, NPT)])
  plsc.subcore_barrier()

  @pl.loop(0, NCHUNK)
  def _(j):
    pltpu.sync_copy(g_hbm.at[src_v.at[j]], rows_v)
    pltpu.sync_copy(rows_v, acc.at[dst_v.at[j]], add=True)

  plsc.subcore_barrier()
  pltpu.sync_copy(acc.at[pl.ds(s * NPT, NPT)], out_hbm.at[w])


def _gelu(z):
  return 0.5 * z * (1.0 + lax.erf(z * 0.7071067811865476))


def _tc_matmul_body(x_ref, w_ref, o_ref):
  o_ref[...] = jnp.dot(x_ref[...], w_ref[...], preferred_element_type=_f32)


def _tc_matmul(x, w):
  return pl.pallas_call(
      _tc_matmul_body,
      grid=(NBLK,),
      in_specs=[
          pl.BlockSpec((BLK, D), lambda i: (i, 0)),
          pl.BlockSpec((D, D), lambda i: (0, 0)),
      ],
      out_specs=pl.BlockSpec((BLK, D), lambda i: (i, 0)),
      out_shape=jax.ShapeDtypeStruct((NP, D), _f32),
  )(x, w)


def _tc_dinv_body(h_ref, r_ref, o_ref):
  hsum = jnp.sum(h_ref[...], axis=0)
  degp = jnp.dot(hsum, r_ref[...], preferred_element_type=_f32)
  o_ref[...] = lax.rsqrt(degp + 1.0)


def _tc_dinv(hr, rmat):
  return pl.pallas_call(
      _tc_dinv_body,
      grid=(H8 // 1024,),
      in_specs=[
          pl.BlockSpec((NW, 8, 128), lambda i: (0, i, 0)),
          pl.BlockSpec((128, 16), lambda i: (0, 0)),
      ],
      out_specs=pl.BlockSpec((8, 16), lambda i: (i, 0)),
      out_shape=jax.ShapeDtypeStruct((H8 // 128, 16), _f32),
  )(hr, rmat)


def _tc_scale_body(dinv_ref, h_ref, o_ref):
  o_ref[...] = dinv_ref[...] * h_ref[...]


def _tc_scale(dinv_col, h):
  return pl.pallas_call(
      _tc_scale_body,
      grid=(NBLK,),
      in_specs=[
          pl.BlockSpec((BLK, 1), lambda i: (i, 0)),
          pl.BlockSpec((BLK, D), lambda i: (i, 0)),
      ],
      out_specs=pl.BlockSpec((BLK, D), lambda i: (i, 0)),
      out_shape=jax.ShapeDtypeStruct((NP, D), _f32),
  )(dinv_col, h)


def _tc_mid_body(sa_ref, sb_ref, g_ref, dinv_ref, b_ref, w_ref, o_ref):
  dinv = dinv_ref[...]
  z = dinv * (sa_ref[...] + sb_ref[...] + g_ref[...]) + b_ref[...]
  a = _gelu(z)
  o_ref[...] = dinv * jnp.dot(a, w_ref[...], preferred_element_type=_f32)


def _tc_mid(sa, sb, g, dinv_col, b_row, w):
  return pl.pallas_call(
      _tc_mid_body,
      grid=(NBLK,),
      in_specs=[
          pl.BlockSpec((BLK, D), lambda i: (i, 0)),
          pl.BlockSpec((BLK, D), lambda i: (i, 0)),
          pl.BlockSpec((BLK, D), lambda i: (i, 0)),
          pl.BlockSpec((BLK, 1), lambda i: (i, 0)),
          pl.BlockSpec((1, D), lambda i: (0, 0)),
          pl.BlockSpec((D, D), lambda i: (0, 0)),
      ],
      out_specs=pl.BlockSpec((BLK, D), lambda i: (i, 0)),
      out_shape=jax.ShapeDtypeStruct((NP, D), _f32),
  )(sa, sb, g, dinv_col, b_row, w)


def _tc_final_body(sa_ref, sb_ref, g_ref, dinv_ref, b_ref, batch_ref,
                   wfc_ref, bfc_ref, o_ref, sums_acc, cnt_acc):
  i = pl.program_id(0)

  @pl.when(i == 0)
  def _():
    sums_acc[...] = jnp.zeros_like(sums_acc)
    cnt_acc[...] = jnp.zeros_like(cnt_acc)

  dinv = dinv_ref[...]
  z = dinv * (sa_ref[...] + sb_ref[...] + g_ref[...]) + b_ref[...]
  a = _gelu(z)

  bl = batch_ref[...].reshape(1, BLK)
  gids = lax.broadcasted_iota(_i32, (G, 1), 0)
  m = (bl == gids).astype(_f32)
  sums_acc[...] += jnp.dot(m, a, preferred_element_type=_f32)
  cnt_acc[...] += jnp.sum(m, axis=1, keepdims=True)

  @pl.when(i == NBLK - 1)
  def _():
    sums = sums_acc[...]
    cnt = jnp.maximum(cnt_acc[...], 1.0)
    o_ref[...] = (jnp.dot(sums, wfc_ref[...], preferred_element_type=_f32)
                  / cnt + bfc_ref[...])


def _tc_final(sa, sb, g, dinv_col, b_row, batch3d, wfc, bfc_row):
  return pl.pallas_call(
      _tc_final_body,
      grid=(NBLK,),
      in_specs=[
          pl.BlockSpec((BLK, D), lambda i: (i, 0)),
          pl.BlockSpec((BLK, D), lambda i: (i, 0)),
          pl.BlockSpec((BLK, D), lambda i: (i, 0)),
          pl.BlockSpec((BLK, 1), lambda i: (i, 0)),
          pl.BlockSpec((1, D), lambda i: (0, 0)),
          pl.BlockSpec((1, 1, BLK), lambda i: (i, 0, 0)),
          pl.BlockSpec((D, C), lambda i: (0, 0)),
          pl.BlockSpec((1, C), lambda i: (0, 0)),
      ],
      out_specs=pl.BlockSpec((G, C), lambda i: (0, 0)),
      out_shape=jax.ShapeDtypeStruct((G, C), _f32),
      scratch_shapes=[
          pltpu.VMEM((G, D), _f32),
          pltpu.VMEM((G, 1), _f32),
      ],
  )(sa, sb, g, dinv_col, b_row, batch3d, wfc, bfc_row)


def kernel(x, edge_index, batch, W1, b1, W2, b2, Wfc, bfc):
  xp = jnp.pad(x.astype(_f32), ((0, NP - N), (0, 0)))
  srcp = jnp.pad(edge_index[0].reshape(NW, EPW),
                 ((0, 0), (0, EPWP - EPW))).reshape(NW, NCHUNK, CH)
  dstp = jnp.pad(edge_index[1].reshape(NW, EPW),
                 ((0, 0), (0, EPWP - EPW)),
                 constant_values=NP - 1).reshape(NW, NCHUNK, CH)
  batch3d = jnp.pad(batch.astype(_i32), (0, NP - N),
                    constant_values=G).reshape(NBLK, 1, BLK)

  lane = jnp.arange(16, dtype=_i32)
  aux = jnp.stack([lane, lane % 8])
  zD = jnp.zeros((NPT, D), _f32)
  hist = _sc_degree(aux, dstp)
  h1 = _tc_matmul(xp, W1.astype(_f32))

  hr = hist.reshape(NW, H8 // 128, 128)
  rmat = (jnp.arange(128, dtype=_i32)[:, None] // 8
          == jnp.arange(16, dtype=_i32)[None, :]).astype(_f32)
  dinv2d = _tc_dinv(hr, rmat)
  dinv_col = dinv2d.reshape(NP, 1)

  g1 = _tc_scale(dinv_col, h1)
  s1 = _sc_aggregate(g1, zD, srcp, dstp).reshape(NC, NP, D)
  g2 = _tc_mid(s1[0], s1[1], g1, dinv_col, b1.reshape(1, D), W2.astype(_f32))
  s2 = _sc_aggregate(g2, zD, srcp, dstp).reshape(NC, NP, D)
  logits = _tc_final(s2[0], s2[1], g2, dinv_col, b2.reshape(1, D),
                     batch3d, Wfc.astype(_f32), bfc.reshape(1, C))
  return logits


# speedup vs baseline: 1.3760x; 1.3760x over previous
import functools

import jax
import jax.numpy as jnp
from jax import lax
from jax.experimental import pallas as pl
from jax.experimental.pallas import tpu as pltpu
from jax.experimental.pallas import tpu_sc as plsc

N = 10000
E = 320000
D = 128
G = 64
C = 10

NC = 2
NS = 16
NW = NC * NS
EPW = E // NW
CH = 128
EPWP = 10112
NCHUNK = EPWP // CH

NP = 10240
NPT = NP // NS
BLK = 512
NBLK = NP // BLK

_f32 = jnp.float32
_i32 = jnp.int32

_MESH = plsc.VectorSubcoreMesh(core_axis_name="c", subcore_axis_name="s")

_SC_PARAMS = pltpu.CompilerParams()
if "needs_layout_passes" in pltpu.CompilerParams.__dataclass_fields__:
  import dataclasses as _dataclasses
  _SC_PARAMS = _dataclasses.replace(_SC_PARAMS, needs_layout_passes=False)


H8 = NP * 8


@functools.partial(
    pl.kernel,
    out_type=jax.ShapeDtypeStruct((NW, H8), _f32),
    mesh=_MESH,
    scratch_types=[
        pltpu.VMEM((NCHUNK, CH), _i32),
        pltpu.VMEM((2, 16), _i32),
        pltpu.VMEM((H8,), _f32),
    ],
    compiler_params=_SC_PARAMS,
)
def _sc_degree(aux_hbm, dstr_hbm, out_hbm, dst_v, aux_v, hist_v):
  c = lax.axis_index("c")
  s = lax.axis_index("s")
  w = c * NS + s
  pltpu.sync_copy(dstr_hbm.at[w], dst_v)
  pltpu.sync_copy(aux_hbm, aux_v)

  zeros16 = jnp.zeros((16,), _f32)
  ones16 = jnp.full((16,), 1.0, _f32)

  @pl.loop(0, H8 // 16)
  def _(r):
    hist_v[pl.ds(r * 16, 16)] = zeros16

  lane = aux_v[0]
  offs = aux_v[1]
  mask_lo = lane < 8
  mask_hi = lane >= 8

  @pl.loop(0, NCHUNK)
  def _(j):
    @pl.loop(0, CH // 16)
    def _(l):
      d = dst_v[j, pl.ds(l * 16, 16)]
      addr = d * 8 + offs
      plsc.addupdate_scatter(hist_v, [addr], ones16, mask=mask_lo)
      plsc.addupdate_scatter(hist_v, [addr], ones16, mask=mask_hi)

  pltpu.sync_copy(hist_v, out_hbm.at[w])


@functools.partial(
    pl.kernel,
    out_type=jax.ShapeDtypeStruct((NW, NPT, D), _f32),
    mesh=_MESH,
    scratch_types=[
        pltpu.VMEM((NCHUNK, CH), _i32),
        pltpu.VMEM((NCHUNK, CH), _i32),
        pltpu.VMEM((CH, D), _f32),
        pltpu.VMEM_SHARED((NP, D), _f32),
    ],
)
def _sc_aggregate(g_hbm, z_hbm, srcr_hbm, dstr_hbm, out_hbm, src_v, dst_v,
                  rows_v, acc):
  c = lax.axis_index("c")
  s = lax.axis_index("s")
  w = c * NS + s
  pltpu.sync_copy(srcr_hbm.at[w], src_v)
  pltpu.sync_copy(dstr_hbm.at[w], dst_v)

  pltpu.sync_copy(z_hbm, acc.at[pl.ds(s * NPT, NPT)])
  plsc.subcore_barrier()

  @pl.loop(0, NCHUNK)
  def _(j):
    pltpu.sync_copy(g_hbm.at[src_v.at[j]], rows_v)
    pltpu.sync_copy(rows_v, acc.at[dst_v.at[j]], add=True)

  plsc.subcore_barrier()
  pltpu.sync_copy(acc.at[pl.ds(s * NPT, NPT)], out_hbm.at[w])


def _gelu(z):
  return 0.5 * z * (1.0 + lax.erf(z * 0.7071067811865476))


def _tc_matmul_body(x_ref, w_ref, o_ref):
  o_ref[...] = jnp.dot(x_ref[...], w_ref[...], preferred_element_type=_f32)


def _tc_matmul(x, w):
  return pl.pallas_call(
      _tc_matmul_body,
      grid=(NBLK,),
      in_specs=[
          pl.BlockSpec((BLK, D), lambda i: (i, 0)),
          pl.BlockSpec((D, D), lambda i: (0, 0)),
      ],
      out_specs=pl.BlockSpec((BLK, D), lambda i: (i, 0)),
      out_shape=jax.ShapeDtypeStruct((NP, D), _f32),
  )(x, w)


def _tc_dinv_body(h_ref, r_ref, o_ref):
  hsum = jnp.sum(h_ref[...], axis=0)
  degp = jnp.dot(hsum, r_ref[...], preferred_element_type=_f32)
  o_ref[...] = lax.rsqrt(degp + 1.0)


def _tc_dinv(hr, rmat):
  return pl.pallas_call(
      _tc_dinv_body,
      grid=(H8 // 1024,),
      in_specs=[
          pl.BlockSpec((NW, 8, 128), lambda i: (0, i, 0)),
          pl.BlockSpec((128, 16), lambda i: (0, 0)),
      ],
      out_specs=pl.BlockSpec((8, 16), lambda i: (i, 0)),
      out_shape=jax.ShapeDtypeStruct((H8 // 128, 16), _f32),
  )(hr, rmat)


def _tc_scale_body(dinv_ref, h_ref, o_ref):
  o_ref[...] = dinv_ref[...] * h_ref[...]


def _tc_scale(dinv_col, h):
  return pl.pallas_call(
      _tc_scale_body,
      grid=(NBLK,),
      in_specs=[
          pl.BlockSpec((BLK, 1), lambda i: (i, 0)),
          pl.BlockSpec((BLK, D), lambda i: (i, 0)),
      ],
      out_specs=pl.BlockSpec((BLK, D), lambda i: (i, 0)),
      out_shape=jax.ShapeDtypeStruct((NP, D), _f32),
  )(dinv_col, h)


def _tc_mid_body(sa_ref, sb_ref, g_ref, dinv_ref, b_ref, w_ref, o_ref):
  dinv = dinv_ref[...]
  z = dinv * (sa_ref[...] + sb_ref[...] + g_ref[...]) + b_ref[...]
  a = _gelu(z)
  o_ref[...] = dinv * jnp.dot(a, w_ref[...], preferred_element_type=_f32)


def _tc_mid(sa, sb, g, dinv_col, b_row, w):
  return pl.pallas_call(
      _tc_mid_body,
      grid=(NBLK,),
      in_specs=[
          pl.BlockSpec((BLK, D), lambda i: (i, 0)),
          pl.BlockSpec((BLK, D), lambda i: (i, 0)),
          pl.BlockSpec((BLK, D), lambda i: (i, 0)),
          pl.BlockSpec((BLK, 1), lambda i: (i, 0)),
          pl.BlockSpec((1, D), lambda i: (0, 0)),
          pl.BlockSpec((D, D), lambda i: (0, 0)),
      ],
      out_specs=pl.BlockSpec((BLK, D), lambda i: (i, 0)),
      out_shape=jax.ShapeDtypeStruct((NP, D), _f32),
  )(sa, sb, g, dinv_col, b_row, w)


def _tc_final_body(sa_ref, sb_ref, g_ref, dinv_ref, b_ref, batch_ref,
                   wfc_ref, bfc_ref, o_ref, sums_acc, cnt_acc):
  i = pl.program_id(0)

  @pl.when(i == 0)
  def _():
    sums_acc[...] = jnp.zeros_like(sums_acc)
    cnt_acc[...] = jnp.zeros_like(cnt_acc)

  dinv = dinv_ref[...]
  z = dinv * (sa_ref[...] + sb_ref[...] + g_ref[...]) + b_ref[...]
  a = _gelu(z)

  bl = batch_ref[...].reshape(1, BLK)
  gids = lax.broadcasted_iota(_i32, (G, 1), 0)
  m = (bl == gids).astype(_f32)
  sums_acc[...] += jnp.dot(m, a, preferred_element_type=_f32)
  cnt_acc[...] += jnp.sum(m, axis=1, keepdims=True)

  @pl.when(i == NBLK - 1)
  def _():
    sums = sums_acc[...]
    cnt = jnp.maximum(cnt_acc[...], 1.0)
    o_ref[...] = (jnp.dot(sums, wfc_ref[...], preferred_element_type=_f32)
                  / cnt + bfc_ref[...])


def _tc_final(sa, sb, g, dinv_col, b_row, batch3d, wfc, bfc_row):
  return pl.pallas_call(
      _tc_final_body,
      grid=(NBLK,),
      in_specs=[
          pl.BlockSpec((BLK, D), lambda i: (i, 0)),
          pl.BlockSpec((BLK, D), lambda i: (i, 0)),
          pl.BlockSpec((BLK, D), lambda i: (i, 0)),
          pl.BlockSpec((BLK, 1), lambda i: (i, 0)),
          pl.BlockSpec((1, D), lambda i: (0, 0)),
          pl.BlockSpec((1, 1, BLK), lambda i: (i, 0, 0)),
          pl.BlockSpec((D, C), lambda i: (0, 0)),
          pl.BlockSpec((1, C), lambda i: (0, 0)),
      ],
      out_specs=pl.BlockSpec((G, C), lambda i: (0, 0)),
      out_shape=jax.ShapeDtypeStruct((G, C), _f32),
      scratch_shapes=[
          pltpu.VMEM((G, D), _f32),
          pltpu.VMEM((G, 1), _f32),
      ],
  )(sa, sb, g, dinv_col, b_row, batch3d, wfc, bfc_row)


def kernel(x, edge_index, batch, W1, b1, W2, b2, Wfc, bfc):
  xp = jnp.pad(x.astype(_f32), ((0, NP - N), (0, 0)))
  srcp = jnp.pad(edge_index[0].reshape(NW, EPW),
                 ((0, 0), (0, EPWP - EPW))).reshape(NW, NCHUNK, CH)
  dummy_dst = (N + jnp.arange(EPWP - EPW, dtype=_i32) % (NP - N))
  dstp = jnp.concatenate(
      [edge_index[1].reshape(NW, EPW),
       jnp.broadcast_to(dummy_dst, (NW, EPWP - EPW))],
      axis=1).reshape(NW, NCHUNK, CH)
  batch3d = jnp.pad(batch.astype(_i32), (0, NP - N),
                    constant_values=G).reshape(NBLK, 1, BLK)

  lane = jnp.arange(16, dtype=_i32)
  aux = jnp.stack([lane, lane % 8])
  zD = jnp.zeros((NPT, D), _f32)
  hist = _sc_degree(aux, dstp)
  h1 = _tc_matmul(xp, W1.astype(_f32))

  hr = hist.reshape(NW, H8 // 128, 128)
  rmat = (jnp.arange(128, dtype=_i32)[:, None] // 8
          == jnp.arange(16, dtype=_i32)[None, :]).astype(_f32)
  dinv2d = _tc_dinv(hr, rmat)
  dinv_col = dinv2d.reshape(NP, 1)

  g1 = _tc_scale(dinv_col, h1)
  s1 = _sc_aggregate(g1, zD, srcp, dstp).reshape(NC, NP, D)
  g2 = _tc_mid(s1[0], s1[1], g1, dinv_col, b1.reshape(1, D), W2.astype(_f32))
  s2 = _sc_aggregate(g2, zD, srcp, dstp).reshape(NC, NP, D)
  logits = _tc_final(s2[0], s2[1], g2, dinv_col, b2.reshape(1, D),
                     batch3d, Wfc.astype(_f32), bfc.reshape(1, C))
  return logits


# speedup vs baseline: 1.3851x; 1.0067x over previous
import functools

import jax
import jax.numpy as jnp
from jax import lax
from jax.experimental import pallas as pl
from jax.experimental.pallas import tpu as pltpu
from jax.experimental.pallas import tpu_sc as plsc

N = 10000
E = 320000
D = 128
G = 64
C = 10

NC = 2
NS = 16
NW = NC * NS
EPW = E // NW
CH = 128
EPWP = 10112
NCHUNK = EPWP // CH

NP = 10240
NPT = NP // NS
BLK = 512
NBLK = NP // BLK

_f32 = jnp.float32
_i32 = jnp.int32

_MESH = plsc.VectorSubcoreMesh(core_axis_name="c", subcore_axis_name="s")

_SC_PARAMS = pltpu.CompilerParams()
if "needs_layout_passes" in pltpu.CompilerParams.__dataclass_fields__:
  import dataclasses as _dataclasses
  _SC_PARAMS = _dataclasses.replace(_SC_PARAMS, needs_layout_passes=False)


H8 = NP * 8


@functools.partial(
    pl.kernel,
    out_type=jax.ShapeDtypeStruct((NW, H8), _f32),
    mesh=_MESH,
    scratch_types=[
        pltpu.VMEM((NCHUNK, CH), _i32),
        pltpu.VMEM((2, 16), _i32),
        pltpu.VMEM((H8,), _f32),
    ],
    compiler_params=_SC_PARAMS,
)
def _sc_degree(aux_hbm, dstr_hbm, out_hbm, dst_v, aux_v, hist_v):
  c = lax.axis_index("c")
  s = lax.axis_index("s")
  w = c * NS + s
  pltpu.sync_copy(dstr_hbm.at[w], dst_v)
  pltpu.sync_copy(aux_hbm, aux_v)

  zeros16 = jnp.zeros((16,), _f32)
  ones16 = jnp.full((16,), 1.0, _f32)

  @pl.loop(0, H8 // 16)
  def _(r):
    hist_v[pl.ds(r * 16, 16)] = zeros16

  lane = aux_v[0]
  offs = aux_v[1]
  mask_lo = lane < 8
  mask_hi = lane >= 8

  @pl.loop(0, NCHUNK)
  def _(j):
    @pl.loop(0, CH // 16)
    def _(l):
      d = dst_v[j, pl.ds(l * 16, 16)]
      addr = d * 8 + offs
      plsc.addupdate_scatter(hist_v, [addr], ones16, mask=mask_lo)
      plsc.addupdate_scatter(hist_v, [addr], ones16, mask=mask_hi)

  pltpu.sync_copy(hist_v, out_hbm.at[w])


@functools.partial(
    pl.kernel,
    out_type=jax.ShapeDtypeStruct((NC, NP, D), _f32),
    mesh=_MESH,
    scratch_types=[
        pltpu.VMEM((NCHUNK, CH), _i32),
        pltpu.VMEM((NCHUNK, CH), _i32),
        pltpu.VMEM((CH, D), _f32),
        pltpu.VMEM_SHARED((NP, D), _f32),
    ],
)
def _sc_aggregate(g_hbm, z_hbm, srcr_hbm, dstr_hbm, out_hbm, src_v, dst_v,
                  rows_v, acc):
  c = lax.axis_index("c")
  s = lax.axis_index("s")
  w = c * NS + s
  pltpu.sync_copy(srcr_hbm.at[w], src_v)
  pltpu.sync_copy(dstr_hbm.at[w], dst_v)

  @pl.when(s == 0)
  def _():
    pltpu.sync_copy(z_hbm, acc)

  plsc.subcore_barrier()

  @pl.loop(0, NCHUNK)
  def _(j):
    pltpu.sync_copy(g_hbm.at[src_v.at[j]], rows_v)
    pltpu.sync_copy(rows_v, acc.at[dst_v.at[j]], add=True)

  plsc.subcore_barrier()

  @pl.when(s == 0)
  def _():
    pltpu.sync_copy(acc, out_hbm.at[c])


def _gelu(z):
  return 0.5 * z * (1.0 + lax.erf(z * 0.7071067811865476))


def _tc_matmul_body(x_ref, w_ref, o_ref):
  o_ref[...] = jnp.dot(x_ref[...], w_ref[...], preferred_element_type=_f32)


def _tc_matmul(x, w):
  return pl.pallas_call(
      _tc_matmul_body,
      grid=(NBLK,),
      in_specs=[
          pl.BlockSpec((BLK, D), lambda i: (i, 0)),
          pl.BlockSpec((D, D), lambda i: (0, 0)),
      ],
      out_specs=pl.BlockSpec((BLK, D), lambda i: (i, 0)),
      out_shape=jax.ShapeDtypeStruct((NP, D), _f32),
  )(x, w)


def _tc_dinv_body(h_ref, r_ref, o_ref):
  hsum = jnp.sum(h_ref[...], axis=0)
  degp = jnp.dot(hsum, r_ref[...], preferred_element_type=_f32)
  o_ref[...] = lax.rsqrt(degp + 1.0)


def _tc_dinv(hr, rmat):
  return pl.pallas_call(
      _tc_dinv_body,
      grid=(H8 // 1024,),
      in_specs=[
          pl.BlockSpec((NW, 8, 128), lambda i: (0, i, 0)),
          pl.BlockSpec((128, 16), lambda i: (0, 0)),
      ],
      out_specs=pl.BlockSpec((8, 16), lambda i: (i, 0)),
      out_shape=jax.ShapeDtypeStruct((H8 // 128, 16), _f32),
  )(hr, rmat)


def _tc_scale_body(dinv_ref, h_ref, o_ref):
  o_ref[...] = dinv_ref[...] * h_ref[...]


def _tc_scale(dinv_col, h):
  return pl.pallas_call(
      _tc_scale_body,
      grid=(NBLK,),
      in_specs=[
          pl.BlockSpec((BLK, 1), lambda i: (i, 0)),
          pl.BlockSpec((BLK, D), lambda i: (i, 0)),
      ],
      out_specs=pl.BlockSpec((BLK, D), lambda i: (i, 0)),
      out_shape=jax.ShapeDtypeStruct((NP, D), _f32),
  )(dinv_col, h)


def _tc_mid_body(sa_ref, sb_ref, g_ref, dinv_ref, b_ref, w_ref, o_ref):
  dinv = dinv_ref[...]
  z = dinv * (sa_ref[...] + sb_ref[...] + g_ref[...]) + b_ref[...]
  a = _gelu(z)
  o_ref[...] = dinv * jnp.dot(a, w_ref[...], preferred_element_type=_f32)


def _tc_mid(sa, sb, g, dinv_col, b_row, w):
  return pl.pallas_call(
      _tc_mid_body,
      grid=(NBLK,),
      in_specs=[
          pl.BlockSpec((BLK, D), lambda i: (i, 0)),
          pl.BlockSpec((BLK, D), lambda i: (i, 0)),
          pl.BlockSpec((BLK, D), lambda i: (i, 0)),
          pl.BlockSpec((BLK, 1), lambda i: (i, 0)),
          pl.BlockSpec((1, D), lambda i: (0, 0)),
          pl.BlockSpec((D, D), lambda i: (0, 0)),
      ],
      out_specs=pl.BlockSpec((BLK, D), lambda i: (i, 0)),
      out_shape=jax.ShapeDtypeStruct((NP, D), _f32),
  )(sa, sb, g, dinv_col, b_row, w)


def _tc_final_body(sa_ref, sb_ref, g_ref, dinv_ref, b_ref, batch_ref,
                   wfc_ref, bfc_ref, o_ref, sums_acc, cnt_acc):
  i = pl.program_id(0)

  @pl.when(i == 0)
  def _():
    sums_acc[...] = jnp.zeros_like(sums_acc)
    cnt_acc[...] = jnp.zeros_like(cnt_acc)

  dinv = dinv_ref[...]
  z = dinv * (sa_ref[...] + sb_ref[...] + g_ref[...]) + b_ref[...]
  a = _gelu(z)

  bl = batch_ref[...].reshape(1, BLK)
  gids = lax.broadcasted_iota(_i32, (G, 1), 0)
  m = (bl == gids).astype(_f32)
  sums_acc[...] += jnp.dot(m, a, preferred_element_type=_f32)
  cnt_acc[...] += jnp.sum(m, axis=1, keepdims=True)

  @pl.when(i == NBLK - 1)
  def _():
    sums = sums_acc[...]
    cnt = jnp.maximum(cnt_acc[...], 1.0)
    o_ref[...] = (jnp.dot(sums, wfc_ref[...], preferred_element_type=_f32)
                  / cnt + bfc_ref[...])


def _tc_final(sa, sb, g, dinv_col, b_row, batch3d, wfc, bfc_row):
  return pl.pallas_call(
      _tc_final_body,
      grid=(NBLK,),
      in_specs=[
          pl.BlockSpec((BLK, D), lambda i: (i, 0)),
          pl.BlockSpec((BLK, D), lambda i: (i, 0)),
          pl.BlockSpec((BLK, D), lambda i: (i, 0)),
          pl.BlockSpec((BLK, 1), lambda i: (i, 0)),
          pl.BlockSpec((1, D), lambda i: (0, 0)),
          pl.BlockSpec((1, 1, BLK), lambda i: (i, 0, 0)),
          pl.BlockSpec((D, C), lambda i: (0, 0)),
          pl.BlockSpec((1, C), lambda i: (0, 0)),
      ],
      out_specs=pl.BlockSpec((G, C), lambda i: (0, 0)),
      out_shape=jax.ShapeDtypeStruct((G, C), _f32),
      scratch_shapes=[
          pltpu.VMEM((G, D), _f32),
          pltpu.VMEM((G, 1), _f32),
      ],
  )(sa, sb, g, dinv_col, b_row, batch3d, wfc, bfc_row)


def kernel(x, edge_index, batch, W1, b1, W2, b2, Wfc, bfc):
  xp = jnp.pad(x.astype(_f32), ((0, NP - N), (0, 0)))
  srcp = jnp.pad(edge_index[0].reshape(NW, EPW),
                 ((0, 0), (0, EPWP - EPW))).reshape(NW, NCHUNK, CH)
  dummy_dst = (N + jnp.arange(EPWP - EPW, dtype=_i32) % (NP - N))
  dstp = jnp.concatenate(
      [edge_index[1].reshape(NW, EPW),
       jnp.broadcast_to(dummy_dst, (NW, EPWP - EPW))],
      axis=1).reshape(NW, NCHUNK, CH)
  batch3d = jnp.pad(batch.astype(_i32), (0, NP - N),
                    constant_values=G).reshape(NBLK, 1, BLK)

  lane = jnp.arange(16, dtype=_i32)
  aux = jnp.stack([lane, lane % 8])
  zD = jnp.zeros((NP, D), _f32)
  hist = _sc_degree(aux, dstp)
  h1 = _tc_matmul(xp, W1.astype(_f32))

  hr = hist.reshape(NW, H8 // 128, 128)
  rmat = (jnp.arange(128, dtype=_i32)[:, None] // 8
          == jnp.arange(16, dtype=_i32)[None, :]).astype(_f32)
  dinv2d = _tc_dinv(hr, rmat)
  dinv_col = dinv2d.reshape(NP, 1)

  g1 = _tc_scale(dinv_col, h1)
  s1 = _sc_aggregate(g1, zD, srcp, dstp)
  g2 = _tc_mid(s1[0], s1[1], g1, dinv_col, b1.reshape(1, D), W2.astype(_f32))
  s2 = _sc_aggregate(g2, zD, srcp, dstp)
  logits = _tc_final(s2[0], s2[1], g2, dinv_col, b2.reshape(1, D),
                     batch3d, Wfc.astype(_f32), bfc.reshape(1, C))
  return logits
